# baseline (speedup 1.0000x reference)
import functools

import jax
import jax.numpy as jnp
from jax import lax
from jax.experimental import pallas as pl
from jax.experimental.pallas import tpu as pltpu
from jax.experimental.pallas import tpu_sc as plsc

_D = 128
_NC = 2
_NS = 16
_NW = _NC * _NS


def _emb_body(rows_per_w, T, group, x_hbm, w_hbm, out_hbm, idx_v, rows_v, g0, g1):
    wid = lax.axis_index("s") * _NC + lax.axis_index("c")
    r0 = wid * rows_per_w
    ngroups = rows_per_w // group

    pltpu.sync_copy(x_hbm.at[pl.ds(r0, rows_per_w)], idx_v)

    sems = (g0, g1)

    def gathers(g, b):
        return [
            pltpu.make_async_copy(
                w_hbm.at[idx_v.at[g * group + j]], rows_v.at[b, j], sems[b])
            for j in range(group)
        ]

    def fire(g, b):
        for c in gathers(g, b):
            c.start()

    def drain(g, b):
        for c in gathers(g, b):
            c.wait()

    def write(g, b):
        pltpu.sync_copy(rows_v.at[b],
                        out_hbm.at[pl.ds(r0 + g * group, group)])

    fire(0, 0)
    fire(1, 1)

    def body(i, carry):
        for b in range(2):
            g = 2 * i + b
            drain(g, b)
            write(g, b)
            fire(g + 2, b)
        return carry

    lax.fori_loop(0, ngroups // 2 - 1, body, 0)

    for b in range(2):
        g = ngroups - 2 + b
        drain(g, b)
        write(g, b)


def kernel(x, weight):
    S, T = x.shape
    nsplit = 2
    Sc = S // nsplit
    rows_per_w = Sc // _NW
    group = 8
    xi = x.astype(jnp.int32)

    mesh = plsc.VectorSubcoreMesh(core_axis_name="c", subcore_axis_name="s")
    k = pl.kernel(
        functools.partial(_emb_body, rows_per_w, T, group),
        out_type=jax.ShapeDtypeStruct((Sc, T, _D), jnp.float32),
        mesh=mesh,
        scratch_types=[
            pltpu.VMEM((rows_per_w, T), jnp.int32),
            pltpu.VMEM((2, group, T, _D), jnp.float32),
            pltpu.SemaphoreType.DMA,
            pltpu.SemaphoreType.DMA,
        ],
    )
    parts = [k(xi[i * Sc:(i + 1) * Sc], weight) for i in range(nsplit)]
    return jnp.concatenate(parts, axis=0)


# speedup vs baseline: 1.6138x; 1.6138x over previous
import functools

import jax
import jax.numpy as jnp
from jax import lax
from jax.experimental import pallas as pl
from jax.experimental.pallas import tpu as pltpu
from jax.experimental.pallas import tpu_sc as plsc

_D = 128
_NC = 2
_NS = 16
_NW = _NC * _NS


def _emb_body(rows_per_w, T, group, x_hbm, w_hbm, out_hbm, idx_v, rows_v, g0, g1):
    wid = lax.axis_index("s") * _NC + lax.axis_index("c")
    r0 = wid * rows_per_w
    ngroups = rows_per_w // group

    pltpu.sync_copy(x_hbm.at[pl.ds(r0, rows_per_w)], idx_v)

    sems = (g0, g1)

    def gathers(g, b):
        return [
            pltpu.make_async_copy(
                w_hbm.at[idx_v.at[g * group + j]], rows_v.at[b, j], sems[b])
            for j in range(group)
        ]

    def fire(g, b):
        for c in gathers(g, b):
            c.start()

    def drain(g, b):
        for c in gathers(g, b):
            c.wait()

    def write(g, b):
        pltpu.sync_copy(rows_v.at[b],
                        out_hbm.at[pl.ds(r0 + g * group, group)])

    fire(0, 0)
    fire(1, 1)

    def body(i, carry):
        for b in range(2):
            g = 2 * i + b
            drain(g, b)
            write(g, b)
            fire(g + 2, b)
        return carry

    lax.fori_loop(0, ngroups // 2 - 1, body, 0)

    for b in range(2):
        g = ngroups - 2 + b
        drain(g, b)
        write(g, b)


def kernel(x, weight):
    S, T = x.shape
    rows_per_w = S // _NW
    group = 8
    xi = x.astype(jnp.int32)

    mesh = plsc.VectorSubcoreMesh(core_axis_name="c", subcore_axis_name="s")
    k = pl.kernel(
        functools.partial(_emb_body, rows_per_w, T, group),
        out_type=jax.ShapeDtypeStruct((S, T, _D), jnp.float32),
        mesh=mesh,
        compiler_params=pltpu.CompilerParams(
            use_tc_tiling_on_sc=True, needs_layout_passes=True),
        scratch_types=[
            pltpu.VMEM((rows_per_w, T), jnp.int32),
            pltpu.VMEM((2, group, T, _D), jnp.float32),
            pltpu.SemaphoreType.DMA,
            pltpu.SemaphoreType.DMA,
        ],
    )
    return k(xi, weight)


# speedup vs baseline: 2.6325x; 1.6312x over previous
import functools

import jax
import jax.numpy as jnp
from jax import lax
from jax.experimental import pallas as pl
from jax.experimental.pallas import tpu as pltpu
from jax.experimental.pallas import tpu_sc as plsc

_D = 128
_NC = 2
_NS = 16
_NW = _NC * _NS


def _emb_body(rows_per_w, T, group, x_hbm, w_hbm, out_hbm, idx_v, col_v, g0, g1):
    wid = lax.axis_index("s") * _NC + lax.axis_index("c")
    r0 = wid * rows_per_w
    ngroups = rows_per_w // group

    pltpu.sync_copy(x_hbm.at[pl.ds(r0, rows_per_w)], idx_v)

    sems = (g0, g1)

    def gathers(g, b):
        return [
            pltpu.make_async_copy(
                w_hbm.at[idx_v.at[g * group + j]],
                col_v.at[b, :, j],
                sems[b])
            for j in range(group)
        ]

    def fire(g, b):
        for c in gathers(g, b):
            c.start()

    def drain(g, b):
        for c in gathers(g, b):
            c.wait()

    def write(g, b):
        pltpu.sync_copy(col_v.at[b],
                        out_hbm.at[:, pl.ds(r0 + g * group, group)])

    fire(0, 0)
    fire(1, 1)

    def body(i, carry):
        for b in range(2):
            g = 2 * i + b
            drain(g, b)
            write(g, b)
            fire(g + 2, b)
        return carry

    lax.fori_loop(0, ngroups // 2 - 1, body, 0)

    for b in range(2):
        g = ngroups - 2 + b
        drain(g, b)
        write(g, b)


def kernel(x, weight):
    S, T = x.shape
    rows_per_w = S // _NW
    group = 8
    xi = x.astype(jnp.int32)

    mesh = plsc.VectorSubcoreMesh(core_axis_name="c", subcore_axis_name="s")
    k = pl.kernel(
        functools.partial(_emb_body, rows_per_w, T, group),
        out_type=jax.ShapeDtypeStruct((T, S, _D), jnp.float32),
        mesh=mesh,
        scratch_types=[
            pltpu.VMEM((rows_per_w, T), jnp.int32),
            pltpu.VMEM((2, T, group, _D), jnp.float32),
            pltpu.SemaphoreType.DMA,
            pltpu.SemaphoreType.DMA,
        ],
    )
    out_t = k(xi, weight)
    return jnp.transpose(out_t, (1, 0, 2))


# speedup vs baseline: 2.8737x; 1.0916x over previous
import functools

import jax
import jax.numpy as jnp
from jax import lax
from jax.experimental import pallas as pl
from jax.experimental.pallas import tpu as pltpu
from jax.experimental.pallas import tpu_sc as plsc

_D = 128
_NC = 2
_NS = 16
_NW = _NC * _NS
_G = 3


def _emb_body(T, iblk, x_hbm, w_hbm, out_hbm, idx_v, buf_v, g0, g1):
    wid = lax.axis_index("s") * _NC + lax.axis_index("c")
    i0 = wid * iblk
    nmain = T // _G
    tail = T - nmain * _G

    pltpu.sync_copy(x_hbm.at[:, pl.ds(i0, iblk)], idx_v)

    sems = (g0, g1)

    def gathers(u, b, n=_G):
        return [
            pltpu.make_async_copy(
                w_hbm.at[idx_v.at[u * _G + j]], buf_v.at[b, j], sems[b])
            for j in range(n)
        ]

    def fire(u, b, n=_G):
        for c in gathers(u, b, n):
            c.start()

    def drain(u, b, n=_G):
        for c in gathers(u, b, n):
            c.wait()

    def write(u, b, n=_G):
        pltpu.sync_copy(buf_v.at[b, pl.ds(0, n)],
                        out_hbm.at[pl.ds(u * _G, n), pl.ds(i0, iblk)])

    fire(0, 0)
    fire(1, 1)

    def body(i, carry):
        for b in range(2):
            u = 2 * i + b
            drain(u, b)
            write(u, b)
            fire(u + 2, b)
        return carry

    lax.fori_loop(0, nmain // 2 - 1, body, 0)

    u = nmain - 2
    drain(u, 0)
    write(u, 0)
    fire(nmain, 0, n=tail)
    drain(u + 1, 1)
    write(u + 1, 1)
    drain(nmain, 0, n=tail)
    write(nmain, 0, n=tail)


def kernel(x, weight):
    S, T = x.shape
    iblk = S // _NW
    xt = jnp.transpose(x.astype(jnp.int32))

    mesh = plsc.VectorSubcoreMesh(core_axis_name="c", subcore_axis_name="s")
    k = pl.kernel(
        functools.partial(_emb_body, T, iblk),
        out_type=jax.ShapeDtypeStruct((T, S, _D), jnp.float32),
        mesh=mesh,
        scratch_types=[
            pltpu.VMEM((T, iblk), jnp.int32),
            pltpu.VMEM((2, _G, iblk, _D), jnp.float32),
            pltpu.SemaphoreType.DMA,
            pltpu.SemaphoreType.DMA,
        ],
    )
    out_t = k(xt, weight)
    return jnp.transpose(out_t, (1, 0, 2))
